# baseline (speedup 1.0000x reference)
import jax
import jax.numpy as jnp
from jax.experimental import pallas as pl
from jax.experimental.pallas import tpu as pltpu

_B, _N, _L = 4, 4096, 1024
_R = _B * _N
_CH = 256
_MAXT = 1024
_RS = 7
_WS = 5
_SCHED = [256, 256, 512] + [1024] * 14 + [512, 256, 256]
assert sum(_SCHED) == _R


def _scan_tile(in_buf, out_buf, rslot, wslot, rows, carry):
    rows_i = jax.lax.broadcasted_iota(jnp.int32, (_CH, _CH), 0)
    cols_i = jax.lax.broadcasted_iota(jnp.int32, (_CH, _CH), 1)
    tri = (cols_i < rows_i).astype(jnp.float32)
    for c in range(rows // _CH):
        xc = in_buf[rslot, pl.ds(c * _CH, _CH), :]
        excl = jnp.dot(tri, xc, preferred_element_type=jnp.float32)
        out_buf[wslot, pl.ds(c * _CH, _CH), :] = excl + carry
        carry = carry + jnp.sum(xc, axis=0, keepdims=True)
    return carry


def _body(x_ref, o_ref, in_buf, out_buf, rsem, wsem):
    T = len(_SCHED)
    starts = [0]
    for r in _SCHED:
        starts.append(starts[-1] + r)

    def rd(t):
        return pltpu.make_async_copy(
            x_ref.at[pl.ds(starts[t], _SCHED[t]), :],
            in_buf.at[t % _RS, pl.ds(0, _SCHED[t]), :],
            rsem.at[t % _RS],
        )

    def wr(t):
        return pltpu.make_async_copy(
            out_buf.at[t % _WS, pl.ds(0, _SCHED[t]), :],
            o_ref.at[pl.ds(starts[t], _SCHED[t]), :],
            wsem.at[t % _WS],
        )

    for t in range(min(_RS, T)):
        rd(t).start()
    carry = jnp.zeros((1, _L), jnp.float32)
    for t in range(T):
        rd(t).wait()
        if t >= _WS:
            wr(t - _WS).wait()
        if starts[t] % _N == 0:
            carry = jnp.zeros((1, _L), jnp.float32)
        carry = _scan_tile(in_buf, out_buf, t % _RS, t % _WS, _SCHED[t], carry)
        wr(t).start()
        if t + _RS < T:
            rd(t + _RS).start()
    for t in range(max(T - _WS, 0), T):
        wr(t).wait()


def kernel(x):
    x2 = x.reshape(_R, _L)
    out = pl.pallas_call(
        _body,
        in_specs=[pl.BlockSpec(memory_space=pl.ANY)],
        out_specs=pl.BlockSpec(memory_space=pl.ANY),
        out_shape=jax.ShapeDtypeStruct((_R, _L), jnp.float32),
        scratch_shapes=[
            pltpu.VMEM((_RS, _MAXT, _L), jnp.float32),
            pltpu.VMEM((_WS, _MAXT, _L), jnp.float32),
            pltpu.SemaphoreType.DMA((_RS,)),
            pltpu.SemaphoreType.DMA((_WS,)),
        ],
    )(x2)
    return out.reshape(_B, _N, _L)


# speedup vs baseline: 1.0015x; 1.0015x over previous
import jax
import jax.numpy as jnp
from jax.experimental import pallas as pl
from jax.experimental.pallas import tpu as pltpu

_B, _N, _L = 4, 4096, 1024
_R = _B * _N
_CH = 256
_MAXT = 2048
_RS = 4
_WS = 3
_SCHED = [256, 256, 512, 1024] + [2048] * 6 + [1024, 512, 256, 256]
assert sum(_SCHED) == _R


def _scan_tile(in_buf, out_buf, rslot, wslot, rows, carry):
    rows_i = jax.lax.broadcasted_iota(jnp.int32, (_CH, _CH), 0)
    cols_i = jax.lax.broadcasted_iota(jnp.int32, (_CH, _CH), 1)
    tri = (cols_i < rows_i).astype(jnp.float32)
    for c in range(rows // _CH):
        xc = in_buf[rslot, pl.ds(c * _CH, _CH), :]
        excl = jnp.dot(tri, xc, preferred_element_type=jnp.float32)
        out_buf[wslot, pl.ds(c * _CH, _CH), :] = excl + carry
        carry = carry + jnp.sum(xc, axis=0, keepdims=True)
    return carry


def _body(x_ref, o_ref, in_buf, out_buf, rsem, wsem):
    T = len(_SCHED)
    starts = [0]
    for r in _SCHED:
        starts.append(starts[-1] + r)

    def rd(t):
        return pltpu.make_async_copy(
            x_ref.at[pl.ds(starts[t], _SCHED[t]), :],
            in_buf.at[t % _RS, pl.ds(0, _SCHED[t]), :],
            rsem.at[t % _RS],
        )

    def wr(t):
        return pltpu.make_async_copy(
            out_buf.at[t % _WS, pl.ds(0, _SCHED[t]), :],
            o_ref.at[pl.ds(starts[t], _SCHED[t]), :],
            wsem.at[t % _WS],
        )

    for t in range(min(_RS, T)):
        rd(t).start()
    carry = jnp.zeros((1, _L), jnp.float32)
    for t in range(T):
        rd(t).wait()
        if t >= _WS:
            wr(t - _WS).wait()
        if starts[t] % _N == 0:
            carry = jnp.zeros((1, _L), jnp.float32)
        carry = _scan_tile(in_buf, out_buf, t % _RS, t % _WS, _SCHED[t], carry)
        wr(t).start()
        if t + _RS < T:
            rd(t + _RS).start()
    for t in range(max(T - _WS, 0), T):
        wr(t).wait()


def kernel(x):
    x2 = x.reshape(_R, _L)
    out = pl.pallas_call(
        _body,
        in_specs=[pl.BlockSpec(memory_space=pl.ANY)],
        out_specs=pl.BlockSpec(memory_space=pl.ANY),
        out_shape=jax.ShapeDtypeStruct((_R, _L), jnp.float32),
        scratch_shapes=[
            pltpu.VMEM((_RS, _MAXT, _L), jnp.float32),
            pltpu.VMEM((_WS, _MAXT, _L), jnp.float32),
            pltpu.SemaphoreType.DMA((_RS,)),
            pltpu.SemaphoreType.DMA((_WS,)),
        ],
    )(x2)
    return out.reshape(_B, _N, _L)


# speedup vs baseline: 1.0031x; 1.0016x over previous
import jax
import jax.numpy as jnp
from jax.experimental import pallas as pl
from jax.experimental.pallas import tpu as pltpu

_B, _N, _L = 4, 4096, 1024
_R = _B * _N
_CH = 256
_MAXT = 2048
_RS = 4
_WS = 3
_SCHED = [2048] * 8
assert sum(_SCHED) == _R


def _scan_tile(in_buf, out_buf, rslot, wslot, rows, carry):
    rows_i = jax.lax.broadcasted_iota(jnp.int32, (_CH, _CH), 0)
    cols_i = jax.lax.broadcasted_iota(jnp.int32, (_CH, _CH), 1)
    tri = (cols_i < rows_i).astype(jnp.float32)
    for c in range(rows // _CH):
        xc = in_buf[rslot, pl.ds(c * _CH, _CH), :]
        excl = jnp.dot(tri, xc, preferred_element_type=jnp.float32)
        out_buf[wslot, pl.ds(c * _CH, _CH), :] = excl + carry
        carry = carry + jnp.sum(xc, axis=0, keepdims=True)
    return carry


def _body(x_ref, o_ref, in_buf, out_buf, rsem, wsem):
    T = len(_SCHED)
    starts = [0]
    for r in _SCHED:
        starts.append(starts[-1] + r)

    def rd(t):
        return pltpu.make_async_copy(
            x_ref.at[pl.ds(starts[t], _SCHED[t]), :],
            in_buf.at[t % _RS, pl.ds(0, _SCHED[t]), :],
            rsem.at[t % _RS],
        )

    def wr(t):
        return pltpu.make_async_copy(
            out_buf.at[t % _WS, pl.ds(0, _SCHED[t]), :],
            o_ref.at[pl.ds(starts[t], _SCHED[t]), :],
            wsem.at[t % _WS],
        )

    for t in range(min(_RS, T)):
        rd(t).start()
    carry = jnp.zeros((1, _L), jnp.float32)
    for t in range(T):
        rd(t).wait()
        if t >= _WS:
            wr(t - _WS).wait()
        if starts[t] % _N == 0:
            carry = jnp.zeros((1, _L), jnp.float32)
        carry = _scan_tile(in_buf, out_buf, t % _RS, t % _WS, _SCHED[t], carry)
        wr(t).start()
        if t + _RS < T:
            rd(t + _RS).start()
    for t in range(max(T - _WS, 0), T):
        wr(t).wait()


def kernel(x):
    x2 = x.reshape(_R, _L)
    out = pl.pallas_call(
        _body,
        in_specs=[pl.BlockSpec(memory_space=pl.ANY)],
        out_specs=pl.BlockSpec(memory_space=pl.ANY),
        out_shape=jax.ShapeDtypeStruct((_R, _L), jnp.float32),
        scratch_shapes=[
            pltpu.VMEM((_RS, _MAXT, _L), jnp.float32),
            pltpu.VMEM((_WS, _MAXT, _L), jnp.float32),
            pltpu.SemaphoreType.DMA((_RS,)),
            pltpu.SemaphoreType.DMA((_WS,)),
        ],
    )(x2)
    return out.reshape(_B, _N, _L)


# speedup vs baseline: 1.0068x; 1.0037x over previous
import jax
import jax.numpy as jnp
from jax.experimental import pallas as pl
from jax.experimental.pallas import tpu as pltpu

_B, _N, _L = 4, 4096, 1024
_R = _B * _N
_CH = 256
_MAXT = 2048
_RS = 4
_WS = 3
_SCHED = [512, 1536] + [2048] * 6 + [1536, 512]
assert sum(_SCHED) == _R


def _scan_tile(in_buf, out_buf, rslot, wslot, rows, carry):
    rows_i = jax.lax.broadcasted_iota(jnp.int32, (_CH, _CH), 0)
    cols_i = jax.lax.broadcasted_iota(jnp.int32, (_CH, _CH), 1)
    tri = (cols_i < rows_i).astype(jnp.float32)
    for c in range(rows // _CH):
        xc = in_buf[rslot, pl.ds(c * _CH, _CH), :]
        excl = jnp.dot(tri, xc, preferred_element_type=jnp.float32)
        out_buf[wslot, pl.ds(c * _CH, _CH), :] = excl + carry
        carry = carry + jnp.sum(xc, axis=0, keepdims=True)
    return carry


def _body(x_ref, o_ref, in_buf, out_buf, rsem, wsem):
    T = len(_SCHED)
    starts = [0]
    for r in _SCHED:
        starts.append(starts[-1] + r)

    def rd(t):
        return pltpu.make_async_copy(
            x_ref.at[pl.ds(starts[t], _SCHED[t]), :],
            in_buf.at[t % _RS, pl.ds(0, _SCHED[t]), :],
            rsem.at[t % _RS],
        )

    def wr(t):
        return pltpu.make_async_copy(
            out_buf.at[t % _WS, pl.ds(0, _SCHED[t]), :],
            o_ref.at[pl.ds(starts[t], _SCHED[t]), :],
            wsem.at[t % _WS],
        )

    for t in range(min(_RS, T)):
        rd(t).start()
    carry = jnp.zeros((1, _L), jnp.float32)
    for t in range(T):
        rd(t).wait()
        if t >= _WS:
            wr(t - _WS).wait()
        if starts[t] % _N == 0:
            carry = jnp.zeros((1, _L), jnp.float32)
        carry = _scan_tile(in_buf, out_buf, t % _RS, t % _WS, _SCHED[t], carry)
        wr(t).start()
        if t + _RS < T:
            rd(t + _RS).start()
    for t in range(max(T - _WS, 0), T):
        wr(t).wait()


def kernel(x):
    x2 = x.reshape(_R, _L)
    out = pl.pallas_call(
        _body,
        in_specs=[pl.BlockSpec(memory_space=pl.ANY)],
        out_specs=pl.BlockSpec(memory_space=pl.ANY),
        out_shape=jax.ShapeDtypeStruct((_R, _L), jnp.float32),
        scratch_shapes=[
            pltpu.VMEM((_RS, _MAXT, _L), jnp.float32),
            pltpu.VMEM((_WS, _MAXT, _L), jnp.float32),
            pltpu.SemaphoreType.DMA((_RS,)),
            pltpu.SemaphoreType.DMA((_WS,)),
        ],
    )(x2)
    return out.reshape(_B, _N, _L)


# speedup vs baseline: 1.0090x; 1.0022x over previous
import jax
import jax.numpy as jnp
from jax.experimental import pallas as pl
from jax.experimental.pallas import tpu as pltpu

_B, _N, _L = 4, 4096, 1024
_R = _B * _N
_CH = 256
_MAXT = 2048
_RS = 4
_WS = 3
_SCHED = [512, 512, 1024] + [2048] * 6 + [1024, 512, 512]
assert sum(_SCHED) == _R


def _scan_tile(in_buf, out_buf, rslot, wslot, rows, carry):
    rows_i = jax.lax.broadcasted_iota(jnp.int32, (_CH, _CH), 0)
    cols_i = jax.lax.broadcasted_iota(jnp.int32, (_CH, _CH), 1)
    tri = (cols_i < rows_i).astype(jnp.float32)
    for c in range(rows // _CH):
        xc = in_buf[rslot, pl.ds(c * _CH, _CH), :]
        excl = jnp.dot(tri, xc, preferred_element_type=jnp.float32)
        out_buf[wslot, pl.ds(c * _CH, _CH), :] = excl + carry
        carry = carry + jnp.sum(xc, axis=0, keepdims=True)
    return carry


def _body(x_ref, o_ref, in_buf, out_buf, rsem, wsem):
    T = len(_SCHED)
    starts = [0]
    for r in _SCHED:
        starts.append(starts[-1] + r)

    def rd(t):
        return pltpu.make_async_copy(
            x_ref.at[pl.ds(starts[t], _SCHED[t]), :],
            in_buf.at[t % _RS, pl.ds(0, _SCHED[t]), :],
            rsem.at[t % _RS],
        )

    def wr(t):
        return pltpu.make_async_copy(
            out_buf.at[t % _WS, pl.ds(0, _SCHED[t]), :],
            o_ref.at[pl.ds(starts[t], _SCHED[t]), :],
            wsem.at[t % _WS],
        )

    for t in range(min(_RS, T)):
        rd(t).start()
    carry = jnp.zeros((1, _L), jnp.float32)
    for t in range(T):
        rd(t).wait()
        if t >= _WS:
            wr(t - _WS).wait()
        if starts[t] % _N == 0:
            carry = jnp.zeros((1, _L), jnp.float32)
        carry = _scan_tile(in_buf, out_buf, t % _RS, t % _WS, _SCHED[t], carry)
        wr(t).start()
        if t + _RS < T:
            rd(t + _RS).start()
    for t in range(max(T - _WS, 0), T):
        wr(t).wait()


def kernel(x):
    x2 = x.reshape(_R, _L)
    out = pl.pallas_call(
        _body,
        in_specs=[pl.BlockSpec(memory_space=pl.ANY)],
        out_specs=pl.BlockSpec(memory_space=pl.ANY),
        out_shape=jax.ShapeDtypeStruct((_R, _L), jnp.float32),
        scratch_shapes=[
            pltpu.VMEM((_RS, _MAXT, _L), jnp.float32),
            pltpu.VMEM((_WS, _MAXT, _L), jnp.float32),
            pltpu.SemaphoreType.DMA((_RS,)),
            pltpu.SemaphoreType.DMA((_WS,)),
        ],
    )(x2)
    return out.reshape(_B, _N, _L)
